# baseline (speedup 1.0000x reference)
import jax
import jax.numpy as jnp
from jax import lax
from jax.experimental import pallas as pl
from jax.experimental.pallas import tpu as pltpu
from jax.experimental.pallas import tpu_sc as plsc

N = 262144
NUM_CORES = 2
NUM_SUBCORES = 16
LANES = 16
NW = NUM_CORES * NUM_SUBCORES
CHUNK = N // NW
NBUF = 4
CSZ = CHUNK // NBUF
UNROLL = 8
STEPS = CSZ // LANES // UNROLL


def _body(pred_hbm, targ_hbm, out_hbm, pred_v, targ_v, out_v, sem):
    cid = lax.axis_index("c")
    sid = lax.axis_index("s")
    wid = sid * NUM_CORES + cid
    base = wid * CHUNK

    copies = []
    for j in range(NBUF):
        copies.append(pltpu.async_copy(
            pred_hbm.at[pl.ds(base + j * CSZ, CSZ)],
            pred_v.at[pl.ds(j * CSZ, CSZ)], sem))
        copies.append(pltpu.async_copy(
            targ_hbm.at[pl.ds(base + j * CSZ, CSZ)],
            targ_v.at[pl.ds(j * CSZ, CSZ)], sem))

    one = jnp.full((LANES,), 1.0, jnp.float32)
    zero = jnp.full((LANES,), 0.0, jnp.float32)

    def make_step(j):
        def step(k, carry):
            ss, cs = carry
            off = j * CSZ + k * (UNROLL * LANES)
            new_ss, new_cs = [], []
            for u in range(UNROLL):
                p = pred_v[pl.ds(off + u * LANES, LANES)]
                t = targ_v[pl.ds(off + u * LANES, LANES)]
                d = p - t
                ok = (d > -0.5) & ((d <= 0.5) | (t == 99.0))
                new_ss.append(ss[u] + jnp.abs(d))
                new_cs.append(cs[u] + jnp.where(ok, one, zero))
            return tuple(new_ss), tuple(new_cs)
        return step

    carry = (tuple(zero for _ in range(UNROLL)),
             tuple(zero for _ in range(UNROLL)))
    for j in range(NBUF):
        copies[2 * j].wait()
        copies[2 * j + 1].wait()
        carry = lax.fori_loop(0, STEPS, make_step(j), carry)

    ss, cs = carry
    s = ss[0]
    c = cs[0]
    for u in range(1, UNROLL):
        s = s + ss[u]
        c = c + cs[u]
    out_v[0, :] = s
    out_v[1, :] = c
    pltpu.sync_copy(out_v.at[0], out_hbm.at[0, pl.ds(wid * LANES, LANES)])
    pltpu.sync_copy(out_v.at[1], out_hbm.at[1, pl.ds(wid * LANES, LANES)])


@jax.jit
def _focus_metrics(pred_flat, target):
    out = pl.kernel(
        _body,
        out_type=jax.ShapeDtypeStruct((2, NW * LANES), jnp.float32),
        mesh=plsc.VectorSubcoreMesh(
            core_axis_name="c", subcore_axis_name="s",
            num_cores=NUM_CORES, num_subcores=NUM_SUBCORES,
        ),
        scratch_types=[
            pltpu.VMEM((CHUNK,), jnp.float32),
            pltpu.VMEM((CHUNK,), jnp.float32),
            pltpu.VMEM((2, LANES), jnp.float32),
            pltpu.SemaphoreType.DMA,
        ],
    )(pred_flat, target)
    totals = jnp.sum(out, axis=1)
    l1 = totals[0] / jnp.float32(N)
    correct = totals[1].astype(jnp.int32)
    total = jnp.array(N, dtype=jnp.int32)
    return l1, correct, total


def kernel(prediction, target):
    return _focus_metrics(prediction.reshape(-1), target)


# speedup vs baseline: 1.0185x; 1.0185x over previous
import jax
import jax.numpy as jnp
from jax import lax
from jax.experimental import pallas as pl
from jax.experimental.pallas import tpu as pltpu
from jax.experimental.pallas import tpu_sc as plsc

N = 262144
NUM_CORES = 1
NUM_SUBCORES = 16
LANES = 16
NW = NUM_CORES * NUM_SUBCORES
CHUNK = N // NW
NBUF = 4
CSZ = CHUNK // NBUF
UNROLL = 8
STEPS = CSZ // LANES // UNROLL


def _body(pred_hbm, targ_hbm, out_hbm, pred_v, targ_v, out_v, sem):
    cid = lax.axis_index("c")
    sid = lax.axis_index("s")
    wid = sid * NUM_CORES + cid
    base = wid * CHUNK

    copies = []
    for j in range(NBUF):
        copies.append(pltpu.async_copy(
            pred_hbm.at[pl.ds(base + j * CSZ, CSZ)],
            pred_v.at[pl.ds(j * CSZ, CSZ)], sem))
        copies.append(pltpu.async_copy(
            targ_hbm.at[pl.ds(base + j * CSZ, CSZ)],
            targ_v.at[pl.ds(j * CSZ, CSZ)], sem))

    one = jnp.full((LANES,), 1.0, jnp.float32)
    zero = jnp.full((LANES,), 0.0, jnp.float32)

    def make_step(j):
        def step(k, carry):
            ss, cs = carry
            off = j * CSZ + k * (UNROLL * LANES)
            new_ss, new_cs = [], []
            for u in range(UNROLL):
                p = pred_v[pl.ds(off + u * LANES, LANES)]
                t = targ_v[pl.ds(off + u * LANES, LANES)]
                d = p - t
                ok = (d > -0.5) & ((d <= 0.5) | (t == 99.0))
                new_ss.append(ss[u] + jnp.abs(d))
                new_cs.append(cs[u] + jnp.where(ok, one, zero))
            return tuple(new_ss), tuple(new_cs)
        return step

    carry = (tuple(zero for _ in range(UNROLL)),
             tuple(zero for _ in range(UNROLL)))
    for j in range(NBUF):
        copies[2 * j].wait()
        copies[2 * j + 1].wait()
        carry = lax.fori_loop(0, STEPS, make_step(j), carry)

    ss, cs = carry
    s = ss[0]
    c = cs[0]
    for u in range(1, UNROLL):
        s = s + ss[u]
        c = c + cs[u]
    out_v[0, :] = s
    out_v[1, :] = c
    pltpu.sync_copy(out_v.at[0], out_hbm.at[0, pl.ds(wid * LANES, LANES)])
    pltpu.sync_copy(out_v.at[1], out_hbm.at[1, pl.ds(wid * LANES, LANES)])


@jax.jit
def _focus_metrics(pred_flat, target):
    out = pl.kernel(
        _body,
        out_type=jax.ShapeDtypeStruct((2, NW * LANES), jnp.float32),
        mesh=plsc.VectorSubcoreMesh(
            core_axis_name="c", subcore_axis_name="s",
            num_cores=NUM_CORES, num_subcores=NUM_SUBCORES,
        ),
        scratch_types=[
            pltpu.VMEM((CHUNK,), jnp.float32),
            pltpu.VMEM((CHUNK,), jnp.float32),
            pltpu.VMEM((2, LANES), jnp.float32),
            pltpu.SemaphoreType.DMA,
        ],
    )(pred_flat, target)
    totals = jnp.sum(out, axis=1)
    l1 = totals[0] / jnp.float32(N)
    correct = totals[1].astype(jnp.int32)
    total = jnp.array(N, dtype=jnp.int32)
    return l1, correct, total


def kernel(prediction, target):
    return _focus_metrics(prediction.reshape(-1), target)
